# Initial kernel scaffold; baseline (speedup 1.0000x reference)
import jax
import jax.numpy as jnp
from jax.experimental import pallas as pl


def kernel(inputs, W):
    raise NotImplementedError("write your pallas kernel here")


# speedup vs baseline: 1.0389x; 1.0389x over previous
import functools

import jax
import jax.numpy as jnp
from jax.experimental import pallas as pl
from jax.experimental.pallas import tpu as pltpu


def _vq_body(x_ref, wt_ref, q_ref, idx_ref, md_ref, sc_ref, hist_ref, acc_ref,
             *, nb, d):
    i = pl.program_id(0)

    @pl.when(i == 0)
    def _init():
        hist_ref[...] = jnp.zeros_like(hist_ref)
        acc_ref[0] = 0.0
        acc_ref[1] = 0.0

    x = x_ref[...]
    wt = wt_ref[...]

    x2 = jnp.sum(x * x, axis=1, keepdims=True)
    w2 = jnp.sum(wt * wt, axis=0, keepdims=True)
    xw = jax.lax.dot_general(x, wt, (((1,), (0,)), ((), ())),
                             preferred_element_type=jnp.float32)
    dist = (x2 + w2) - 2.0 * xw

    md = jnp.min(dist, axis=1, keepdims=True)
    idx = jnp.argmin(dist, axis=1, keepdims=True)

    bm, k = dist.shape
    onehot = (jax.lax.broadcasted_iota(jnp.int32, (bm, k), 1)
              == idx).astype(jnp.float32)
    quant = jax.lax.dot_general(onehot, wt, (((1,), (1,)), ((), ())),
                                preferred_element_type=jnp.float32)

    valid = jnp.sqrt(x2) > 1e-06
    maskf = valid.astype(jnp.float32)

    q_ref[...] = x + (quant * maskf - x)
    idx_ref[...] = jnp.where(valid, idx, 0)
    md_ref[...] = jnp.where(valid, md, 0.0)

    hist_ref[...] += jnp.sum(onehot * maskf, axis=0, keepdims=True)
    acc_ref[0] += jnp.sum(maskf * (quant - x) ** 2)
    acc_ref[1] += jnp.sum(maskf)

    @pl.when(i == nb - 1)
    def _finish():
        n_valid = jnp.maximum(acc_ref[1], 1.0)
        loss_vq = acc_ref[0] / (n_valid * d)
        p = hist_ref[...] / n_valid
        entropy = jnp.sum(p * jnp.log(p + 1e-10))
        perplexity = jnp.exp(-entropy)
        perplexity_loss = -jnp.log(perplexity + 1e-10)
        sc_ref[0] = loss_vq + 0.01 * perplexity_loss
        sc_ref[1] = loss_vq
        sc_ref[2] = perplexity_loss
        sc_ref[3] = perplexity


@functools.partial(jax.jit, static_argnames=())
def kernel(inputs, W):
    d = W.shape[1]
    K = W.shape[0]
    flat = inputs.reshape(-1, d)
    M = flat.shape[0]
    BM = 256
    nb = M // BM

    wt = W.T

    q, idx, md, sc = pl.pallas_call(
        functools.partial(_vq_body, nb=nb, d=d),
        grid=(nb,),
        in_specs=[
            pl.BlockSpec((BM, d), lambda i: (i, 0)),
            pl.BlockSpec((d, K), lambda i: (0, 0)),
        ],
        out_specs=[
            pl.BlockSpec((BM, d), lambda i: (i, 0)),
            pl.BlockSpec((BM, 1), lambda i: (i, 0)),
            pl.BlockSpec((BM, 1), lambda i: (i, 0)),
            pl.BlockSpec(memory_space=pltpu.SMEM),
        ],
        out_shape=[
            jax.ShapeDtypeStruct((M, d), jnp.float32),
            jax.ShapeDtypeStruct((M, 1), jnp.int32),
            jax.ShapeDtypeStruct((M, 1), jnp.float32),
            jax.ShapeDtypeStruct((4,), jnp.float32),
        ],
        scratch_shapes=[
            pltpu.VMEM((1, K), jnp.float32),
            pltpu.SMEM((2,), jnp.float32),
        ],
    )(flat, wt)

    quantized_st = q.reshape(inputs.shape)
    idx_flat = idx.reshape(-1)
    indices = idx.reshape(inputs.shape[:-1])
    min_distances = md.reshape(inputs.shape[:-1])
    return (quantized_st, sc[0], idx_flat, indices, min_distances,
            sc[1], sc[2], sc[3])


# speedup vs baseline: 1.2942x; 1.2458x over previous
import functools

import jax
import jax.numpy as jnp
from jax import lax
from jax.experimental import pallas as pl
from jax.experimental.pallas import tpu as pltpu
from jax.experimental.pallas import tpu_sc as plsc

_NC, _NS = 2, 16
_NW = _NC * _NS
_CHUNK = 128


def _tc_dist_body(x_ref, wt_ref, idxm_ref, idxg_ref, md_ref, sc_ref, acc_ref,
                  *, nb, d, k):
    i = pl.program_id(0)

    @pl.when(i == 0)
    def _init():
        acc_ref[0] = 0.0
        acc_ref[1] = 0.0

    x = x_ref[...]
    wt = wt_ref[...]
    x2 = jnp.sum(x * x, axis=1, keepdims=True)
    w2 = jnp.sum(wt * wt, axis=0, keepdims=True)
    xw = lax.dot_general(x, wt, (((1,), (0,)), ((), ())),
                         preferred_element_type=jnp.float32)
    dist = (x2 + w2) - 2.0 * xw

    md = jnp.min(dist, axis=1, keepdims=True)
    idx = jnp.argmin(dist, axis=1, keepdims=True)
    valid = jnp.sqrt(x2) > 1e-06

    idxm_ref[...] = jnp.where(valid, idx, 0)
    idxg_ref[...] = jnp.where(valid, idx, k)
    mdm = jnp.where(valid, md, 0.0)
    md_ref[...] = mdm

    acc_ref[0] += jnp.sum(mdm)
    acc_ref[1] += jnp.sum(valid.astype(jnp.float32))

    @pl.when(i == nb - 1)
    def _fin():
        nv = jnp.maximum(acc_ref[1], 1.0)
        sc_ref[0] = acc_ref[0] / (nv * d)
        sc_ref[1] = nv


def _sc_gather_body(idxg_hbm, w_hbm, qst_hbm, hist_hbm,
                    idx_v, rows_v, ones_v, zeros_v, hist_sh, sem,
                    *, rpw, hist_pad):
    cid = lax.axis_index("c")
    sid = lax.axis_index("s")
    wid = sid * _NC + cid
    base = wid * rpw
    nchunks = rpw // _CHUNK
    stripe = hist_pad // _NS

    zero16 = jnp.zeros((16,), jnp.float32)

    def _zero(i, carry):
        zeros_v[pl.ds(i * 16, 16)] = zero16
        return carry

    lax.fori_loop(0, stripe // 16, _zero, 0)
    pltpu.sync_copy(zeros_v, hist_sh.at[pl.ds(sid * stripe, stripe)])

    one16 = jnp.ones((16,), jnp.float32)

    def _ones(i, carry):
        ones_v[pl.ds(i * 16, 16)] = one16
        return carry

    lax.fori_loop(0, _CHUNK // 16, _ones, 0)

    for c in range(nchunks):
        pltpu.sync_copy(idxg_hbm.at[pl.ds(base + c * _CHUNK, _CHUNK)],
                        idx_v.at[c])
    for c in range(nchunks):
        pltpu.async_copy(w_hbm.at[idx_v.at[c]],
                         rows_v.at[pl.ds(c * _CHUNK, _CHUNK)], sem).wait()
    pltpu.sync_copy(rows_v, qst_hbm.at[pl.ds(base, rpw)])

    plsc.subcore_barrier()
    for c in range(nchunks):
        pltpu.sync_copy(ones_v, hist_sh.at[idx_v.at[c]], add=True)
    plsc.subcore_barrier()

    @pl.when(sid == 0)
    def _emit():
        pltpu.sync_copy(hist_sh, hist_hbm.at[cid])


def _tc_final_body(hist_ref, sc1_ref, out_ref, *, k):
    counts = jnp.sum(hist_ref[...][:, :k], axis=0, keepdims=True)
    nv = sc1_ref[1]
    p = counts / nv
    entropy = jnp.sum(p * jnp.log(p + 1e-10))
    perplexity = jnp.exp(-entropy)
    perplexity_loss = -jnp.log(perplexity + 1e-10)
    loss_vq = sc1_ref[0]
    out_ref[0] = loss_vq + 0.01 * perplexity_loss
    out_ref[1] = loss_vq
    out_ref[2] = perplexity_loss
    out_ref[3] = perplexity


@jax.jit
def kernel(inputs, W):
    d = W.shape[1]
    K = W.shape[0]
    flat = inputs.reshape(-1, d)
    M = flat.shape[0]
    BM = 256
    nb = M // BM
    rpw = M // _NW
    hist_pad = K + 512

    wt = W.T

    idxm, idxg, md, sc1 = pl.pallas_call(
        functools.partial(_tc_dist_body, nb=nb, d=d, k=K),
        grid=(nb,),
        in_specs=[
            pl.BlockSpec((BM, d), lambda i: (i, 0)),
            pl.BlockSpec((d, K), lambda i: (0, 0)),
        ],
        out_specs=[
            pl.BlockSpec((BM, 1), lambda i: (i, 0)),
            pl.BlockSpec((BM, 1), lambda i: (i, 0)),
            pl.BlockSpec((BM, 1), lambda i: (i, 0)),
            pl.BlockSpec(memory_space=pltpu.SMEM),
        ],
        out_shape=[
            jax.ShapeDtypeStruct((M, 1), jnp.int32),
            jax.ShapeDtypeStruct((M, 1), jnp.int32),
            jax.ShapeDtypeStruct((M, 1), jnp.float32),
            jax.ShapeDtypeStruct((2,), jnp.float32),
        ],
        scratch_shapes=[pltpu.SMEM((2,), jnp.float32)],
    )(flat, wt)

    w_aug = jnp.concatenate([W, jnp.zeros((8, d), jnp.float32)], axis=0)

    qst, hist = pl.kernel(
        functools.partial(_sc_gather_body, rpw=rpw, hist_pad=hist_pad),
        out_type=[
            jax.ShapeDtypeStruct((M, d), jnp.float32),
            jax.ShapeDtypeStruct((_NC, hist_pad), jnp.float32),
        ],
        mesh=plsc.VectorSubcoreMesh(core_axis_name="c", subcore_axis_name="s"),
        scratch_types=[
            pltpu.VMEM((rpw // _CHUNK, _CHUNK), jnp.int32),
            pltpu.VMEM((rpw, d), jnp.float32),
            pltpu.VMEM((_CHUNK,), jnp.float32),
            pltpu.VMEM((hist_pad // _NS,), jnp.float32),
            pltpu.VMEM_SHARED((hist_pad,), jnp.float32),
            pltpu.SemaphoreType.DMA,
        ],
        compiler_params=pltpu.CompilerParams(use_tc_tiling_on_sc=False),
    )(idxg.reshape(-1), w_aug)

    sc3 = pl.pallas_call(
        functools.partial(_tc_final_body, k=K),
        in_specs=[
            pl.BlockSpec((_NC, hist_pad), lambda: (0, 0)),
            pl.BlockSpec(memory_space=pltpu.SMEM),
        ],
        out_specs=pl.BlockSpec(memory_space=pltpu.SMEM),
        out_shape=jax.ShapeDtypeStruct((4,), jnp.float32),
    )(hist, sc1)

    return (qst.reshape(inputs.shape), sc3[0], idxm.reshape(-1),
            idxm.reshape(inputs.shape[:-1]), md.reshape(inputs.shape[:-1]),
            sc3[1], sc3[2], sc3[3])
